# Initial kernel scaffold; baseline (speedup 1.0000x reference)
import jax
import jax.numpy as jnp
from jax.experimental import pallas as pl


def kernel(x, edge_index, W1, b1, W2, b2, W3, b3, fcW1, fcb1, fcW2, fcb2):
    raise NotImplementedError("write your pallas kernel here")


# speedup vs baseline: 4.6592x; 4.6592x over previous
import functools

import jax
import jax.numpy as jnp
from jax import lax
from jax.experimental import pallas as pl
from jax.experimental.pallas import tpu as pltpu
from jax.experimental.pallas import tpu_sc as plsc

N = 10000
NP = 10240
E = 320000
DIN = 128
H = 256
C = 2
K = 5
CH = 128
NCHUNK = E // CH
NSUB = 16
NCORE = 2
ZROWS = NP // NSUB
HI = jax.lax.Precision.HIGHEST


def _mesh():
    return plsc.VectorSubcoreMesh(core_axis_name="c", subcore_axis_name="s")


def _deg(dst_r, zeros128, ones128):
    @functools.partial(
        pl.kernel,
        out_type=jax.ShapeDtypeStruct((NCORE, NP, 128), jnp.float32),
        mesh=_mesh(),
        scratch_types=[
            pltpu.VMEM((1, CH), jnp.int32),
            pltpu.VMEM((CH, 128), jnp.float32),
            pltpu.VMEM_SHARED((NP, 128), jnp.float32),
        ],
    )
    def k(dst_hbm, z_hbm, ones_hbm, out_hbm, didx, ones_v, acc):
        cid = lax.axis_index("c")
        sid = lax.axis_index("s")
        wid = sid * NCORE + cid
        pltpu.sync_copy(z_hbm, acc.at[pl.ds(sid * ZROWS, ZROWS)])
        pltpu.sync_copy(ones_hbm, ones_v)
        plsc.subcore_barrier()
        c0 = wid * NCHUNK // (NCORE * NSUB)
        c1 = (wid + 1) * NCHUNK // (NCORE * NSUB)

        def body(j, carry):
            pltpu.sync_copy(dst_hbm.at[pl.ds(j, 1)], didx)
            pltpu.sync_copy(ones_v, acc.at[didx.at[0]], add=True)
            return carry

        lax.fori_loop(c0, c1, body, 0)
        plsc.subcore_barrier()
        pltpu.sync_copy(acc.at[pl.ds(sid * ZROWS, ZROWS)],
                        out_hbm.at[cid].at[pl.ds(sid * ZROWS, ZROWS)])

    return k(dst_r, zeros128, ones128)


def _spmm(y, edge_r, zeros, split_edges):
    @functools.partial(
        pl.kernel,
        out_type=jax.ShapeDtypeStruct((NCORE, NP, 128), jnp.float32),
        mesh=_mesh(),
        scratch_types=[
            pltpu.VMEM((2, CH), jnp.int32),
            pltpu.VMEM((CH, 128), jnp.float32),
            pltpu.VMEM_SHARED((NP, 128), jnp.float32),
            pltpu.SemaphoreType.DMA,
        ],
    )
    def k(y_hbm, e_hbm, z_hbm, out_hbm, eidx, rows, acc, sem):
        cid = lax.axis_index("c")
        sid = lax.axis_index("s")
        pltpu.sync_copy(z_hbm, acc.at[pl.ds(sid * ZROWS, ZROWS)])
        plsc.subcore_barrier()
        if split_edges:
            half = NCHUNK // NCORE
            c0 = cid * half + sid * half // NSUB
            c1 = cid * half + (sid + 1) * half // NSUB
        else:
            c0 = sid * NCHUNK // NSUB
            c1 = (sid + 1) * NCHUNK // NSUB

        def body(j, carry):
            pltpu.sync_copy(e_hbm.at[j], eidx)
            if split_edges:
                src = y_hbm.at[eidx.at[0]]
            else:
                src = y_hbm.at[cid].at[eidx.at[0]]
            pltpu.async_copy(src, rows, sem).wait()
            pltpu.sync_copy(rows, acc.at[eidx.at[1]], add=True)
            return carry

        lax.fori_loop(c0, c1, body, 0)
        plsc.subcore_barrier()
        pltpu.sync_copy(acc.at[pl.ds(sid * ZROWS, ZROWS)],
                        out_hbm.at[cid].at[pl.ds(sid * ZROWS, ZROWS)])

    return k(y, edge_r, zeros)


def _norm_y0(degp, x):
    TN = 1000

    def body(dp, xr, nrm, y0):
        d = dp[0, :, 0:1] + dp[1, :, 0:1]
        nv = lax.rsqrt(jnp.maximum(d, 1.0))
        nrm[...] = nv
        y0[...] = xr[...] * nv

    return pl.pallas_call(
        body,
        grid=(N // TN,),
        in_specs=[pl.BlockSpec((NCORE, TN, 128), lambda i: (0, i, 0)),
                  pl.BlockSpec((TN, DIN), lambda i: (i, 0))],
        out_specs=[pl.BlockSpec((TN, 1), lambda i: (i, 0)),
                   pl.BlockSpec((TN, DIN), lambda i: (i, 0))],
        out_shape=[jax.ShapeDtypeStruct((N, 1), jnp.float32),
                   jax.ShapeDtypeStruct((N, DIN), jnp.float32)],
    )(degp, x)


def _cheb_l1(hp, norm, yprev, coef, emit_y):
    TN = 1000
    has_prev = yprev is not None

    def body(*refs):
        if has_prev:
            h_ref, n_ref, yp_ref = refs[0], refs[1], refs[2]
            outs = refs[3:]
        else:
            h_ref, n_ref = refs[0], refs[1]
            outs = refs[2:]
        nv = n_ref[...]
        g = (h_ref[0] + h_ref[1]) * nv
        outs[0][...] = g
        if emit_y:
            y = coef * g * nv
            if has_prev:
                y = y - yp_ref[...]
            outs[1][...] = y

    in_specs = [pl.BlockSpec((NCORE, TN, 128), lambda i: (0, i, 0)),
                pl.BlockSpec((TN, 1), lambda i: (i, 0))]
    ins = [hp, norm]
    if has_prev:
        in_specs.append(pl.BlockSpec((TN, 128), lambda i: (i, 0)))
        ins.append(yprev)
    out_specs = [pl.BlockSpec((TN, 128), lambda i: (i, 0))]
    out_shape = [jax.ShapeDtypeStruct((N, 128), jnp.float32)]
    if emit_y:
        out_specs.append(pl.BlockSpec((TN, 128), lambda i: (i, 0)))
        out_shape.append(jax.ShapeDtypeStruct((N, 128), jnp.float32))
    r = pl.pallas_call(body, grid=(N // TN,), in_specs=in_specs,
                       out_specs=out_specs, out_shape=out_shape)(*ins)
    return (r[0], r[1]) if emit_y else (r[0], None)


def _cheb(h, norm, yprev, coef, Fb, emit_y):
    TN = 1000
    F = 2 * Fb
    has_prev = yprev is not None

    def body(*refs):
        if has_prev:
            h_ref, n_ref, yp_ref = refs[0], refs[1], refs[2]
            outs = refs[3:]
        else:
            h_ref, n_ref = refs[0], refs[1]
            outs = refs[2:]
        nv = n_ref[...]
        g0 = h_ref[0] * nv
        g1 = h_ref[1] * nv
        g_ref = outs[0]
        g_ref[:, :Fb] = g0
        g_ref[:, Fb:] = g1
        if emit_y:
            y_ref = outs[1]
            y0 = coef * g0 * nv
            y1 = coef * g1 * nv
            if has_prev:
                y0 = y0 - yp_ref[0]
                y1 = y1 - yp_ref[1]
            y_ref[0] = y0
            y_ref[1] = y1

    in_specs = [pl.BlockSpec((NCORE, TN, Fb), lambda i: (0, i, 0)),
                pl.BlockSpec((TN, 1), lambda i: (i, 0))]
    ins = [h, norm]
    if has_prev:
        in_specs.append(pl.BlockSpec((NCORE, TN, Fb), lambda i: (0, i, 0)))
        ins.append(yprev)
    out_specs = [pl.BlockSpec((TN, F), lambda i: (i, 0))]
    out_shape = [jax.ShapeDtypeStruct((N, F), jnp.float32)]
    if emit_y:
        out_specs.append(pl.BlockSpec((NCORE, TN, Fb), lambda i: (0, i, 0)))
        out_shape.append(jax.ShapeDtypeStruct((NCORE, N, Fb), jnp.float32))
    r = pl.pallas_call(body, grid=(N // TN,), in_specs=in_specs,
                       out_specs=out_specs, out_shape=out_shape)(*ins)
    return (r[0], r[1]) if emit_y else (r[0], None)


def _layer_mm(xin, gs, wm, bias, norm, Fin, final):
    TN = 400
    GN = N // TN

    def body(x_ref, g1_ref, g2_ref, g3_ref, g4_ref, w_ref, b_ref, n_ref, *outs):
        acc = jnp.dot(x_ref[...], w_ref[0], preferred_element_type=jnp.float32,
                      precision=HI)
        for gr, kk in ((g1_ref, 1), (g2_ref, 2), (g3_ref, 3), (g4_ref, 4)):
            acc = acc + jnp.dot(gr[...], w_ref[kk],
                                preferred_element_type=jnp.float32, precision=HI)
        acc = acc + b_ref[...]
        hv = jnp.maximum(acc, 0.0)
        if final:
            psum = jnp.sum(hv, axis=0, keepdims=True)
            row = lax.broadcasted_iota(jnp.int32, (8, 1), 0)
            outs[0][...] = jnp.where(row == 0, psum, 0.0)
        else:
            outs[0][...] = hv
            nv = n_ref[...]
            outs[1][0] = hv[:, :128] * nv
            outs[1][1] = hv[:, 128:] * nv

    in_specs = ([pl.BlockSpec((TN, Fin), lambda i: (i, 0))] * 5 +
                [pl.BlockSpec((K, Fin, H), lambda i: (0, 0, 0)),
                 pl.BlockSpec((1, H), lambda i: (0, 0)),
                 pl.BlockSpec((TN, 1), lambda i: (i, 0))])
    if final:
        out_specs = [pl.BlockSpec((8, H), lambda i: (i, 0))]
        out_shape = [jax.ShapeDtypeStruct((GN * 8, H), jnp.float32)]
    else:
        out_specs = [pl.BlockSpec((TN, H), lambda i: (i, 0)),
                     pl.BlockSpec((NCORE, TN, 128), lambda i: (0, i, 0))]
        out_shape = [jax.ShapeDtypeStruct((N, H), jnp.float32),
                     jax.ShapeDtypeStruct((NCORE, N, 128), jnp.float32)]
    r = pl.pallas_call(body, grid=(GN,), in_specs=in_specs,
                       out_specs=out_specs, out_shape=out_shape)(
        xin, gs[0], gs[1], gs[2], gs[3], wm, bias, norm)
    return r


def _head(partials, fcW1, fcb1, fcW2, fcb2):
    def body(p, w1, b1, w2, b2, o):
        hg = jnp.sum(p[...], axis=0, keepdims=True) * (1.0 / N)
        t = jnp.dot(hg, w1[...], preferred_element_type=jnp.float32,
                    precision=HI) + b1[...]
        t = jnp.maximum(t, 0.0)
        u = jnp.dot(t, w2[...], preferred_element_type=jnp.float32,
                    precision=HI) + b2[...]
        m = jnp.max(u, axis=1, keepdims=True)
        lse = m + jnp.log(jnp.sum(jnp.exp(u - m), axis=1, keepdims=True))
        o[...] = u - lse

    return pl.pallas_call(
        body, out_shape=jax.ShapeDtypeStruct((1, C), jnp.float32),
    )(partials, fcW1, fcb1, fcW2, fcb2)


def _wmod(W, fin):
    Wk = W.reshape(K, fin, -1)
    return jnp.stack([
        Wk[0] - Wk[2] + Wk[4],
        -Wk[1] + Wk[3],
        -2.0 * (Wk[2] - Wk[4]),
        -2.0 * Wk[3],
        -2.0 * Wk[4],
    ])


def kernel(x, edge_index, W1, b1, W2, b2, W3, b3, fcW1, fcb1, fcW2, fcb2):
    f32 = jnp.float32
    edge_r = jnp.transpose(edge_index.reshape(2, NCHUNK, CH), (1, 0, 2))
    dst_r = edge_index[1].reshape(NCHUNK, CH)
    zeros128 = jnp.zeros((ZROWS, 128), f32)
    ones128 = jnp.ones((CH, 128), f32)

    degp = _deg(dst_r, zeros128, ones128)
    norm, y = _norm_y0(degp, x)

    xin = x
    partials = None
    for li, (W, bb) in enumerate(((W1, b1), (W2, b2), (W3, b3))):
        Fin = DIN if li == 0 else H
        first = li == 0
        wm = _wmod(W, Fin)
        gs = []
        yp2 = None
        ycur = y
        for kk in range(1, K):
            hk = _spmm(ycur, edge_r, zeros128, split_edges=first)
            emit = kk < K - 1
            coef = -1.0 if kk == 1 else -2.0
            yprev = yp2 if kk >= 2 else None
            if first:
                g, ynew = _cheb_l1(hk, norm, yprev, coef, emit)
            else:
                g, ynew = _cheb(hk, norm, yprev, coef, 128, emit)
            gs.append(g)
            yp2, ycur = ycur, ynew
        final = li == 2
        if final:
            partials = _layer_mm(xin, gs, wm, bb.reshape(1, -1), norm, Fin,
                                 True)[0]
        else:
            xin, y = _layer_mm(xin, gs, wm, bb.reshape(1, -1), norm, Fin,
                               False)
    return _head(partials, fcW1, fcb1.reshape(1, -1), fcW2, fcb2.reshape(1, -1))


# speedup vs baseline: 7.6583x; 1.6437x over previous
import functools

import jax
import jax.numpy as jnp
from jax import lax
from jax.experimental import pallas as pl
from jax.experimental.pallas import tpu as pltpu
from jax.experimental.pallas import tpu_sc as plsc

N = 10000
NP = 10240
E = 320000
DIN = 128
H = 256
C = 2
K = 5
CH = 128
NCHUNK = E // CH
NSUB = 16
NCORE = 2
ZROWS = NP // NSUB
HI = jax.lax.Precision.HIGHEST


def _mesh():
    return plsc.VectorSubcoreMesh(core_axis_name="c", subcore_axis_name="s")


def _deg(dst_r, zeros128, ones128):
    @functools.partial(
        pl.kernel,
        out_type=jax.ShapeDtypeStruct((NCORE, NP, 128), jnp.float32),
        mesh=_mesh(),
        scratch_types=[
            pltpu.VMEM((1, CH), jnp.int32),
            pltpu.VMEM((CH, 128), jnp.float32),
            pltpu.VMEM_SHARED((NP, 128), jnp.float32),
        ],
    )
    def k(dst_hbm, z_hbm, ones_hbm, out_hbm, didx, ones_v, acc):
        cid = lax.axis_index("c")
        sid = lax.axis_index("s")
        wid = sid * NCORE + cid
        pltpu.sync_copy(z_hbm, acc.at[pl.ds(sid * ZROWS, ZROWS)])
        pltpu.sync_copy(ones_hbm, ones_v)
        plsc.subcore_barrier()
        c0 = wid * NCHUNK // (NCORE * NSUB)
        c1 = (wid + 1) * NCHUNK // (NCORE * NSUB)

        def body(j, carry):
            pltpu.sync_copy(dst_hbm.at[pl.ds(j, 1)], didx)
            pltpu.sync_copy(ones_v, acc.at[didx.at[0]], add=True)
            return carry

        lax.fori_loop(c0, c1, body, 0)
        plsc.subcore_barrier()
        pltpu.sync_copy(acc.at[pl.ds(sid * ZROWS, ZROWS)],
                        out_hbm.at[cid].at[pl.ds(sid * ZROWS, ZROWS)])

    return k(dst_r, zeros128, ones128)


def _spmm(y, edge_r, zeros, split_edges):
    if split_edges:
        per_sub = (NCHUNK // NCORE + NSUB - 1) // NSUB
    else:
        per_sub = (NCHUNK + NSUB - 1) // NSUB
    ngroups = (per_sub + 1) // 2

    @functools.partial(
        pl.kernel,
        out_type=jax.ShapeDtypeStruct((NCORE, NP, 128), jnp.float32),
        mesh=_mesh(),
        scratch_types=[
            pltpu.VMEM((2, CH), jnp.int32),
            pltpu.VMEM((2, CH), jnp.int32),
            pltpu.VMEM((CH, 128), jnp.float32),
            pltpu.VMEM((CH, 128), jnp.float32),
            pltpu.VMEM_SHARED((NP, 128), jnp.float32),
            pltpu.SemaphoreType.DMA,
            pltpu.SemaphoreType.DMA,
        ],
    )
    def k(y_hbm, e_hbm, z_hbm, out_hbm, eidx0, eidx1, rows0, rows1, acc,
          sem0, sem1):
        cid = lax.axis_index("c")
        sid = lax.axis_index("s")
        pltpu.sync_copy(z_hbm, acc.at[pl.ds(sid * ZROWS, ZROWS)])
        plsc.subcore_barrier()
        if split_edges:
            half = NCHUNK // NCORE
            c0 = cid * half + sid * half // NSUB
            c1 = cid * half + (sid + 1) * half // NSUB
        else:
            c0 = sid * NCHUNK // NSUB
            c1 = (sid + 1) * NCHUNK // NSUB
        slots = ((eidx0, rows0, sem0), (eidx1, rows1, sem1))

        def gsrc(eidx):
            if split_edges:
                return y_hbm.at[eidx.at[0]]
            return y_hbm.at[cid].at[eidx.at[0]]

        for b in range(2):
            eidx, rows, sem = slots[b]
            jb = c0 + b

            @pl.when(jb < c1)
            def _():
                pltpu.sync_copy(e_hbm.at[jb], eidx)
                pltpu.async_copy(gsrc(eidx), rows, sem)

        def grp(g, carry):
            for b in range(2):
                eidx, rows, sem = slots[b]
                j = c0 + 2 * g + b

                @pl.when(j < c1)
                def _():
                    pltpu.make_async_copy(gsrc(eidx), rows, sem).wait()
                    pltpu.sync_copy(rows, acc.at[eidx.at[1]], add=True)
                    jn = j + 2

                    @pl.when(jn < c1)
                    def _():
                        pltpu.sync_copy(e_hbm.at[jn], eidx)
                        pltpu.async_copy(gsrc(eidx), rows, sem)
            return carry

        lax.fori_loop(0, ngroups, grp, 0)
        plsc.subcore_barrier()
        pltpu.sync_copy(acc.at[pl.ds(sid * ZROWS, ZROWS)],
                        out_hbm.at[cid].at[pl.ds(sid * ZROWS, ZROWS)])

    return k(y, edge_r, zeros)


def _norm_y0(degp, x):
    TN = 1000

    def body(dp, xr, nrm, y0):
        d = dp[0, :, 0:1] + dp[1, :, 0:1]
        nv = lax.rsqrt(jnp.maximum(d, 1.0))
        nrm[...] = nv
        y0[...] = xr[...] * nv

    return pl.pallas_call(
        body,
        grid=(N // TN,),
        in_specs=[pl.BlockSpec((NCORE, TN, 128), lambda i: (0, i, 0)),
                  pl.BlockSpec((TN, DIN), lambda i: (i, 0))],
        out_specs=[pl.BlockSpec((TN, 1), lambda i: (i, 0)),
                   pl.BlockSpec((TN, DIN), lambda i: (i, 0))],
        out_shape=[jax.ShapeDtypeStruct((N, 1), jnp.float32),
                   jax.ShapeDtypeStruct((N, DIN), jnp.float32)],
    )(degp, x)


def _cheb_l1(hp, norm, yprev, coef, emit_y):
    TN = 1000
    has_prev = yprev is not None

    def body(*refs):
        if has_prev:
            h_ref, n_ref, yp_ref = refs[0], refs[1], refs[2]
            outs = refs[3:]
        else:
            h_ref, n_ref = refs[0], refs[1]
            outs = refs[2:]
        nv = n_ref[...]
        g = (h_ref[0] + h_ref[1]) * nv
        outs[0][...] = g
        if emit_y:
            y = coef * g * nv
            if has_prev:
                y = y - yp_ref[...]
            outs[1][...] = y

    in_specs = [pl.BlockSpec((NCORE, TN, 128), lambda i: (0, i, 0)),
                pl.BlockSpec((TN, 1), lambda i: (i, 0))]
    ins = [hp, norm]
    if has_prev:
        in_specs.append(pl.BlockSpec((TN, 128), lambda i: (i, 0)))
        ins.append(yprev)
    out_specs = [pl.BlockSpec((TN, 128), lambda i: (i, 0))]
    out_shape = [jax.ShapeDtypeStruct((N, 128), jnp.float32)]
    if emit_y:
        out_specs.append(pl.BlockSpec((TN, 128), lambda i: (i, 0)))
        out_shape.append(jax.ShapeDtypeStruct((N, 128), jnp.float32))
    r = pl.pallas_call(body, grid=(N // TN,), in_specs=in_specs,
                       out_specs=out_specs, out_shape=out_shape)(*ins)
    return (r[0], r[1]) if emit_y else (r[0], None)


def _cheb(h, norm, yprev, coef, Fb, emit_y):
    TN = 1000
    F = 2 * Fb
    has_prev = yprev is not None

    def body(*refs):
        if has_prev:
            h_ref, n_ref, yp_ref = refs[0], refs[1], refs[2]
            outs = refs[3:]
        else:
            h_ref, n_ref = refs[0], refs[1]
            outs = refs[2:]
        nv = n_ref[...]
        g0 = h_ref[0] * nv
        g1 = h_ref[1] * nv
        g_ref = outs[0]
        g_ref[:, :Fb] = g0
        g_ref[:, Fb:] = g1
        if emit_y:
            y_ref = outs[1]
            y0 = coef * g0 * nv
            y1 = coef * g1 * nv
            if has_prev:
                y0 = y0 - yp_ref[0]
                y1 = y1 - yp_ref[1]
            y_ref[0] = y0
            y_ref[1] = y1

    in_specs = [pl.BlockSpec((NCORE, TN, Fb), lambda i: (0, i, 0)),
                pl.BlockSpec((TN, 1), lambda i: (i, 0))]
    ins = [h, norm]
    if has_prev:
        in_specs.append(pl.BlockSpec((NCORE, TN, Fb), lambda i: (0, i, 0)))
        ins.append(yprev)
    out_specs = [pl.BlockSpec((TN, F), lambda i: (i, 0))]
    out_shape = [jax.ShapeDtypeStruct((N, F), jnp.float32)]
    if emit_y:
        out_specs.append(pl.BlockSpec((NCORE, TN, Fb), lambda i: (0, i, 0)))
        out_shape.append(jax.ShapeDtypeStruct((NCORE, N, Fb), jnp.float32))
    r = pl.pallas_call(body, grid=(N // TN,), in_specs=in_specs,
                       out_specs=out_specs, out_shape=out_shape)(*ins)
    return (r[0], r[1]) if emit_y else (r[0], None)


def _layer_mm(xin, gs, wm, bias, norm, Fin, final):
    TN = 400
    GN = N // TN

    def body(x_ref, g1_ref, g2_ref, g3_ref, g4_ref, w_ref, b_ref, n_ref, *outs):
        acc = jnp.dot(x_ref[...], w_ref[0], preferred_element_type=jnp.float32,
                      precision=HI)
        for gr, kk in ((g1_ref, 1), (g2_ref, 2), (g3_ref, 3), (g4_ref, 4)):
            acc = acc + jnp.dot(gr[...], w_ref[kk],
                                preferred_element_type=jnp.float32, precision=HI)
        acc = acc + b_ref[...]
        hv = jnp.maximum(acc, 0.0)
        if final:
            psum = jnp.sum(hv, axis=0, keepdims=True)
            row = lax.broadcasted_iota(jnp.int32, (8, 1), 0)
            outs[0][...] = jnp.where(row == 0, psum, 0.0)
        else:
            outs[0][...] = hv
            nv = n_ref[...]
            outs[1][0] = hv[:, :128] * nv
            outs[1][1] = hv[:, 128:] * nv

    in_specs = ([pl.BlockSpec((TN, Fin), lambda i: (i, 0))] * 5 +
                [pl.BlockSpec((K, Fin, H), lambda i: (0, 0, 0)),
                 pl.BlockSpec((1, H), lambda i: (0, 0)),
                 pl.BlockSpec((TN, 1), lambda i: (i, 0))])
    if final:
        out_specs = [pl.BlockSpec((8, H), lambda i: (i, 0))]
        out_shape = [jax.ShapeDtypeStruct((GN * 8, H), jnp.float32)]
    else:
        out_specs = [pl.BlockSpec((TN, H), lambda i: (i, 0)),
                     pl.BlockSpec((NCORE, TN, 128), lambda i: (0, i, 0))]
        out_shape = [jax.ShapeDtypeStruct((N, H), jnp.float32),
                     jax.ShapeDtypeStruct((NCORE, N, 128), jnp.float32)]
    r = pl.pallas_call(body, grid=(GN,), in_specs=in_specs,
                       out_specs=out_specs, out_shape=out_shape)(
        xin, gs[0], gs[1], gs[2], gs[3], wm, bias, norm)
    return r


def _head(partials, fcW1, fcb1, fcW2, fcb2):
    def body(p, w1, b1, w2, b2, o):
        hg = jnp.sum(p[...], axis=0, keepdims=True) * (1.0 / N)
        t = jnp.dot(hg, w1[...], preferred_element_type=jnp.float32,
                    precision=HI) + b1[...]
        t = jnp.maximum(t, 0.0)
        u = jnp.dot(t, w2[...], preferred_element_type=jnp.float32,
                    precision=HI) + b2[...]
        m = jnp.max(u, axis=1, keepdims=True)
        lse = m + jnp.log(jnp.sum(jnp.exp(u - m), axis=1, keepdims=True))
        o[...] = u - lse

    return pl.pallas_call(
        body, out_shape=jax.ShapeDtypeStruct((1, C), jnp.float32),
    )(partials, fcW1, fcb1, fcW2, fcb2)


def _wmod(W, fin):
    Wk = W.reshape(K, fin, -1)
    return jnp.stack([
        Wk[0] - Wk[2] + Wk[4],
        -Wk[1] + Wk[3],
        -2.0 * (Wk[2] - Wk[4]),
        -2.0 * Wk[3],
        -2.0 * Wk[4],
    ])


def kernel(x, edge_index, W1, b1, W2, b2, W3, b3, fcW1, fcb1, fcW2, fcb2):
    f32 = jnp.float32
    edge_r = jnp.transpose(edge_index.reshape(2, NCHUNK, CH), (1, 0, 2))
    dst_r = edge_index[1].reshape(NCHUNK, CH)
    zeros128 = jnp.zeros((ZROWS, 128), f32)
    ones128 = jnp.ones((CH, 128), f32)

    degp = _deg(dst_r, zeros128, ones128)
    norm, y = _norm_y0(degp, x)

    xin = x
    partials = None
    for li, (W, bb) in enumerate(((W1, b1), (W2, b2), (W3, b3))):
        Fin = DIN if li == 0 else H
        first = li == 0
        wm = _wmod(W, Fin)
        gs = []
        yp2 = None
        ycur = y
        for kk in range(1, K):
            hk = _spmm(ycur, edge_r, zeros128, split_edges=first)
            emit = kk < K - 1
            coef = -1.0 if kk == 1 else -2.0
            yprev = yp2 if kk >= 2 else None
            if first:
                g, ynew = _cheb_l1(hk, norm, yprev, coef, emit)
            else:
                g, ynew = _cheb(hk, norm, yprev, coef, 128, emit)
            gs.append(g)
            yp2, ycur = ycur, ynew
        final = li == 2
        if final:
            partials = _layer_mm(xin, gs, wm, bb.reshape(1, -1), norm, Fin,
                                 True)[0]
        else:
            xin, y = _layer_mm(xin, gs, wm, bb.reshape(1, -1), norm, Fin,
                               False)
    return _head(partials, fcW1, fcb1.reshape(1, -1), fcW2, fcb2.reshape(1, -1))


# speedup vs baseline: 9.0899x; 1.1869x over previous
import functools

import jax
import jax.numpy as jnp
from jax import lax
from jax.experimental import pallas as pl
from jax.experimental.pallas import tpu as pltpu
from jax.experimental.pallas import tpu_sc as plsc

N = 10000
NP = 10112
E = 320000
DIN = 128
H = 256
C = 2
K = 5
CH = 112
NCH = -(-E // CH)
TRASH = 10104
NSUB = 16
NCORE = 2
ZROWS = NP // NSUB
RS = 3
SI = 5
UN = 15
HI = jax.lax.Precision.HIGHEST


def _mesh():
    return plsc.VectorSubcoreMesh(core_axis_name="c", subcore_axis_name="s")


def _deg(e_pad, zeros128, ones128):
    NW = NCORE * NSUB

    @functools.partial(
        pl.kernel,
        out_type=jax.ShapeDtypeStruct((NCORE, NP, 128), jnp.float32),
        mesh=_mesh(),
        scratch_types=[
            pltpu.VMEM((2, CH), jnp.int32),
            pltpu.VMEM((2, CH), jnp.int32),
            pltpu.VMEM((CH, 128), jnp.float32),
            pltpu.VMEM_SHARED((NP, 128), jnp.float32),
            pltpu.SemaphoreType.DMA,
            pltpu.SemaphoreType.DMA,
        ],
    )
    def k(e_hbm, z_hbm, ones_hbm, out_hbm, ei0, ei1, ones_v, acc, sm0, sm1):
        cid = lax.axis_index("c")
        sid = lax.axis_index("s")
        wid = sid * NCORE + cid
        c0 = wid * NCH // NW
        c1 = (wid + 1) * NCH // NW
        cnt = c1 - c0
        eidx = (ei0, ei1)
        semi = (sm0, sm1)

        def idx_issue(r, b):
            pltpu.async_copy(e_hbm.at[c0 + r], eidx[b], semi[b])

        def idx_wait(r, b):
            pltpu.make_async_copy(e_hbm.at[c0 + r], eidx[b], semi[b]).wait()

        for b in range(2):
            idx_issue(b, b)
        pltpu.sync_copy(ones_hbm, ones_v)
        pltpu.sync_copy(z_hbm, acc.at[pl.ds(sid * ZROWS, ZROWS)])
        plsc.subcore_barrier()

        def grp(g, carry):
            for b in range(2):
                r = 2 * g + b

                @pl.when(r < cnt)
                def _():
                    idx_wait(r, b)
                    pltpu.sync_copy(ones_v, acc.at[eidx[b].at[1]], add=True)
                    rn = r + 2

                    @pl.when(rn < cnt)
                    def _():
                        idx_issue(rn, b)
            return carry

        lax.fori_loop(0, (cnt + 1) // 2, grp, 0)
        plsc.subcore_barrier()
        pltpu.sync_copy(acc.at[pl.ds(sid * ZROWS, ZROWS)],
                        out_hbm.at[cid].at[pl.ds(sid * ZROWS, ZROWS)])

    return k(e_pad, zeros128, ones128)


def _spmm(y, edge_r, zeros, split_edges):
    total = NCH // NCORE if split_edges else NCH
    per_sub = (total + NSUB - 1) // NSUB
    ngroups = (per_sub + UN - 1) // UN

    @functools.partial(
        pl.kernel,
        out_type=jax.ShapeDtypeStruct((NCORE, NP, 128), jnp.float32),
        mesh=_mesh(),
        scratch_types=(
            [pltpu.VMEM((2, CH), jnp.int32)] * SI
            + [pltpu.VMEM((CH, 128), jnp.float32)] * RS
            + [pltpu.VMEM_SHARED((NP, 128), jnp.float32)]
            + [pltpu.SemaphoreType.DMA] * (SI + 2 * RS)
        ),
    )
    def k(y_hbm, e_hbm, z_hbm, out_hbm, *scr):
        eidx = scr[:SI]
        rows = scr[SI:SI + RS]
        acc = scr[SI + RS]
        semi = scr[SI + RS + 1:SI + RS + 1 + SI]
        semg = scr[SI + RS + 1 + SI:SI + RS + 1 + SI + RS]
        sems = scr[SI + RS + 1 + SI + RS:]
        cid = lax.axis_index("c")
        sid = lax.axis_index("s")
        base = cid * (NCH // NCORE) if split_edges else 0
        c0 = base + sid * total // NSUB
        c1 = base + (sid + 1) * total // NSUB
        cnt = c1 - c0

        def gsrc(bi):
            idx = eidx[bi].at[0]
            if split_edges:
                return y_hbm.at[idx]
            return y_hbm.at[cid].at[idx]

        def idx_issue(r, bi):
            pltpu.async_copy(e_hbm.at[c0 + r], eidx[bi], semi[bi])

        def idx_wait(r, bi):
            pltpu.make_async_copy(e_hbm.at[c0 + r], eidx[bi], semi[bi]).wait()

        def g_issue(bi, br):
            pltpu.async_copy(gsrc(bi), rows[br], semg[br])

        def g_wait(bi, br):
            pltpu.make_async_copy(gsrc(bi), rows[br], semg[br]).wait()

        def s_issue(bi, br):
            pltpu.async_copy(rows[br], acc.at[eidx[bi].at[1]], sems[br],
                             add=True)

        def s_wait(bi, br):
            pltpu.make_async_copy(rows[br], acc.at[eidx[bi].at[1]],
                                  sems[br]).wait()

        for m in range(4):
            @pl.when(m < cnt)
            def _(m=m):
                idx_issue(m, m % SI)
        for m in range(2):
            @pl.when(m < cnt)
            def _(m=m):
                idx_wait(m, m % SI)
                g_issue(m % SI, m % RS)
        pltpu.sync_copy(z_hbm, acc.at[pl.ds(sid * ZROWS, ZROWS)])
        plsc.subcore_barrier()

        def grp(g, carry):
            for u in range(UN):
                r = UN * g + u
                br, bi = u % RS, u % SI
                brn, bin_ = (u + 2) % RS, (u + 2) % SI
                bii = (u + 4) % SI

                @pl.when(r < cnt)
                def _(r=r, br=br, bi=bi, brn=brn, bin_=bin_, bii=bii):
                    g_wait(bi, br)
                    s_issue(bi, br)
                    rn = r + 2

                    @pl.when(rn < cnt)
                    def _():
                        @pl.when(r >= 1)
                        def _():
                            s_wait(bin_, brn)
                        idx_wait(rn, bin_)
                        g_issue(bin_, brn)

                    ri = r + 4

                    @pl.when(ri < cnt)
                    def _():
                        idx_issue(ri, bii)
            return carry

        lax.fori_loop(0, ngroups, grp, 0)
        for br in range(RS):
            s_wait(0, br)
        plsc.subcore_barrier()
        pltpu.sync_copy(acc.at[pl.ds(sid * ZROWS, ZROWS)],
                        out_hbm.at[cid].at[pl.ds(sid * ZROWS, ZROWS)])

    return k(y, edge_r, zeros)


def _norm_y0(degp, x):
    TN = 1000

    def body(dp, xr, nrm, y0):
        d = dp[0, :, 0:1] + dp[1, :, 0:1]
        nv = lax.rsqrt(jnp.maximum(d, 1.0))
        nrm[...] = nv
        y0[...] = xr[...] * nv

    return pl.pallas_call(
        body,
        grid=(N // TN,),
        in_specs=[pl.BlockSpec((NCORE, TN, 128), lambda i: (0, i, 0)),
                  pl.BlockSpec((TN, DIN), lambda i: (i, 0))],
        out_specs=[pl.BlockSpec((TN, 1), lambda i: (i, 0)),
                   pl.BlockSpec((TN, DIN), lambda i: (i, 0))],
        out_shape=[jax.ShapeDtypeStruct((N, 1), jnp.float32),
                   jax.ShapeDtypeStruct((N, DIN), jnp.float32)],
    )(degp, x)


def _cheb_l1(hp, norm, yprev, coef, emit_y):
    TN = 1000
    has_prev = yprev is not None

    def body(*refs):
        if has_prev:
            h_ref, n_ref, yp_ref = refs[0], refs[1], refs[2]
            outs = refs[3:]
        else:
            h_ref, n_ref = refs[0], refs[1]
            outs = refs[2:]
        nv = n_ref[...]
        g = (h_ref[0] + h_ref[1]) * nv
        outs[0][...] = g
        if emit_y:
            y = coef * g * nv
            if has_prev:
                y = y - yp_ref[...]
            outs[1][...] = y

    in_specs = [pl.BlockSpec((NCORE, TN, 128), lambda i: (0, i, 0)),
                pl.BlockSpec((TN, 1), lambda i: (i, 0))]
    ins = [hp, norm]
    if has_prev:
        in_specs.append(pl.BlockSpec((TN, 128), lambda i: (i, 0)))
        ins.append(yprev)
    out_specs = [pl.BlockSpec((TN, 128), lambda i: (i, 0))]
    out_shape = [jax.ShapeDtypeStruct((N, 128), jnp.float32)]
    if emit_y:
        out_specs.append(pl.BlockSpec((TN, 128), lambda i: (i, 0)))
        out_shape.append(jax.ShapeDtypeStruct((N, 128), jnp.float32))
    r = pl.pallas_call(body, grid=(N // TN,), in_specs=in_specs,
                       out_specs=out_specs, out_shape=out_shape)(*ins)
    return (r[0], r[1]) if emit_y else (r[0], None)


def _cheb(h, norm, yprev, coef, Fb, emit_y):
    TN = 1000
    F = 2 * Fb
    has_prev = yprev is not None

    def body(*refs):
        if has_prev:
            h_ref, n_ref, yp_ref = refs[0], refs[1], refs[2]
            outs = refs[3:]
        else:
            h_ref, n_ref = refs[0], refs[1]
            outs = refs[2:]
        nv = n_ref[...]
        g0 = h_ref[0] * nv
        g1 = h_ref[1] * nv
        g_ref = outs[0]
        g_ref[:, :Fb] = g0
        g_ref[:, Fb:] = g1
        if emit_y:
            y_ref = outs[1]
            y0 = coef * g0 * nv
            y1 = coef * g1 * nv
            if has_prev:
                y0 = y0 - yp_ref[0]
                y1 = y1 - yp_ref[1]
            y_ref[0] = y0
            y_ref[1] = y1

    in_specs = [pl.BlockSpec((NCORE, TN, Fb), lambda i: (0, i, 0)),
                pl.BlockSpec((TN, 1), lambda i: (i, 0))]
    ins = [h, norm]
    if has_prev:
        in_specs.append(pl.BlockSpec((NCORE, TN, Fb), lambda i: (0, i, 0)))
        ins.append(yprev)
    out_specs = [pl.BlockSpec((TN, F), lambda i: (i, 0))]
    out_shape = [jax.ShapeDtypeStruct((N, F), jnp.float32)]
    if emit_y:
        out_specs.append(pl.BlockSpec((NCORE, TN, Fb), lambda i: (0, i, 0)))
        out_shape.append(jax.ShapeDtypeStruct((NCORE, N, Fb), jnp.float32))
    r = pl.pallas_call(body, grid=(N // TN,), in_specs=in_specs,
                       out_specs=out_specs, out_shape=out_shape)(*ins)
    return (r[0], r[1]) if emit_y else (r[0], None)


def _layer_mm(xin, gs, wm, bias, norm, Fin, final):
    TN = 400
    GN = N // TN

    def body(x_ref, g1_ref, g2_ref, g3_ref, g4_ref, w_ref, b_ref, n_ref, *outs):
        acc = jnp.dot(x_ref[...], w_ref[0], preferred_element_type=jnp.float32,
                      precision=HI)
        for gr, kk in ((g1_ref, 1), (g2_ref, 2), (g3_ref, 3), (g4_ref, 4)):
            acc = acc + jnp.dot(gr[...], w_ref[kk],
                                preferred_element_type=jnp.float32, precision=HI)
        acc = acc + b_ref[...]
        hv = jnp.maximum(acc, 0.0)
        if final:
            psum = jnp.sum(hv, axis=0, keepdims=True)
            row = lax.broadcasted_iota(jnp.int32, (8, 1), 0)
            outs[0][...] = jnp.where(row == 0, psum, 0.0)
        else:
            outs[0][...] = hv
            nv = n_ref[...]
            outs[1][0] = hv[:, :128] * nv
            outs[1][1] = hv[:, 128:] * nv

    in_specs = ([pl.BlockSpec((TN, Fin), lambda i: (i, 0))] * 5 +
                [pl.BlockSpec((K, Fin, H), lambda i: (0, 0, 0)),
                 pl.BlockSpec((1, H), lambda i: (0, 0)),
                 pl.BlockSpec((TN, 1), lambda i: (i, 0))])
    if final:
        out_specs = [pl.BlockSpec((8, H), lambda i: (i, 0))]
        out_shape = [jax.ShapeDtypeStruct((GN * 8, H), jnp.float32)]
    else:
        out_specs = [pl.BlockSpec((TN, H), lambda i: (i, 0)),
                     pl.BlockSpec((NCORE, TN, 128), lambda i: (0, i, 0))]
        out_shape = [jax.ShapeDtypeStruct((N, H), jnp.float32),
                     jax.ShapeDtypeStruct((NCORE, N, 128), jnp.float32)]
    r = pl.pallas_call(body, grid=(GN,), in_specs=in_specs,
                       out_specs=out_specs, out_shape=out_shape)(
        xin, gs[0], gs[1], gs[2], gs[3], wm, bias, norm)
    return r


def _head(partials, fcW1, fcb1, fcW2, fcb2):
    def body(p, w1, b1, w2, b2, o):
        hg = jnp.sum(p[...], axis=0, keepdims=True) * (1.0 / N)
        t = jnp.dot(hg, w1[...], preferred_element_type=jnp.float32,
                    precision=HI) + b1[...]
        t = jnp.maximum(t, 0.0)
        u = jnp.dot(t, w2[...], preferred_element_type=jnp.float32,
                    precision=HI) + b2[...]
        m = jnp.max(u, axis=1, keepdims=True)
        lse = m + jnp.log(jnp.sum(jnp.exp(u - m), axis=1, keepdims=True))
        o[...] = u - lse

    return pl.pallas_call(
        body, out_shape=jax.ShapeDtypeStruct((1, C), jnp.float32),
    )(partials, fcW1, fcb1, fcW2, fcb2)


def _wmod(W, fin):
    Wk = W.reshape(K, fin, -1)
    return jnp.stack([
        Wk[0] - Wk[2] + Wk[4],
        -Wk[1] + Wk[3],
        -2.0 * (Wk[2] - Wk[4]),
        -2.0 * Wk[3],
        -2.0 * Wk[4],
    ])


def kernel(x, edge_index, W1, b1, W2, b2, W3, b3, fcW1, fcb1, fcW2, fcb2):
    f32 = jnp.float32
    npad = NCH * CH - E
    pad = jnp.concatenate(
        [jnp.zeros((1, npad), jnp.int32),
         jnp.full((1, npad), TRASH, jnp.int32)], axis=0)
    e_pad = jnp.transpose(
        jnp.concatenate([edge_index, pad], axis=1).reshape(2, NCH, CH),
        (1, 0, 2))
    zeros128 = jnp.zeros((ZROWS, 128), f32)
    ones128 = jnp.ones((CH, 128), f32)

    degp = _deg(e_pad, zeros128, ones128)
    norm, y = _norm_y0(degp, x)

    xin = x
    partials = None
    for li, (W, bb) in enumerate(((W1, b1), (W2, b2), (W3, b3))):
        Fin = DIN if li == 0 else H
        first = li == 0
        wm = _wmod(W, Fin)
        gs = []
        yp2 = None
        ycur = y
        for kk in range(1, K):
            hk = _spmm(ycur, e_pad, zeros128, split_edges=first)
            emit = kk < K - 1
            coef = -1.0 if kk == 1 else -2.0
            yprev = yp2 if kk >= 2 else None
            if first:
                g, ynew = _cheb_l1(hk, norm, yprev, coef, emit)
            else:
                g, ynew = _cheb(hk, norm, yprev, coef, 128, emit)
            gs.append(g)
            yp2, ycur = ycur, ynew
        final = li == 2
        if final:
            partials = _layer_mm(xin, gs, wm, bb.reshape(1, -1), norm, Fin,
                                 True)[0]
        else:
            xin, y = _layer_mm(xin, gs, wm, bb.reshape(1, -1), norm, Fin,
                               False)
    return _head(partials, fcW1, fcb1.reshape(1, -1), fcW2, fcb2.reshape(1, -1))
